# baseline (speedup 1.0000x reference)
import functools

import jax
import jax.numpy as jnp
from jax import lax
from jax.experimental import pallas as pl
from jax.experimental.pallas import tpu as pltpu
from jax.experimental.pallas import tpu_sc as plsc

VOCAB = 1000
EMBED_DIM = 64
_PADDED = 128

_NC = 2
_NS = 16
_NW = _NC * _NS

_CHUNK = 128
_K = 4
_SLAB = _K * _CHUNK


def _make_gather(n_idx: int):
    per_w = n_idx // _NW
    chunks_per_w = per_w // _CHUNK
    steps = chunks_per_w // _K
    assert per_w * _NW == n_idx
    assert steps * _K == chunks_per_w and chunks_per_w * _CHUNK == per_w
    assert steps >= 2 and steps % 2 == 0

    mesh = plsc.VectorSubcoreMesh(core_axis_name="c", subcore_axis_name="s")

    @functools.partial(
        pl.kernel,
        mesh=mesh,
        out_type=jax.ShapeDtypeStruct((n_idx, _PADDED), jnp.float32),
        scratch_types=[
            pltpu.VMEM_SHARED((VOCAB, EMBED_DIM), jnp.float32),
            pltpu.VMEM((2 * _K, _CHUNK), jnp.int32),
            pltpu.VMEM((2 * _SLAB, EMBED_DIM), jnp.float32),
            pltpu.SemaphoreType.DMA,
            pltpu.SemaphoreType.DMA,
            pltpu.SemaphoreType.DMA,
        ],
        compiler_params=pltpu.CompilerParams(use_tc_tiling_on_sc=False),
    )
    def k(table_hbm, idx_hbm, out_hbm, table_sh, idx_v, rows_v, gsem, wsem0,
          wsem1):
        cid = lax.axis_index("c")
        sid = lax.axis_index("s")
        wid = sid * _NC + cid

        @pl.when(sid == 0)
        def _stage():
            pltpu.sync_copy(table_hbm, table_sh)

        plsc.subcore_barrier()

        def do_step(g, slot):
            wsem = wsem0 if slot == 0 else wsem1
            cbase = wid * chunks_per_w + g * _K
            pltpu.sync_copy(
                idx_hbm.at[pl.ds(cbase, _K)],
                idx_v.at[pl.ds(slot * _K, _K)],
            )
            copies = [
                pltpu.async_copy(
                    table_sh.at[idx_v.at[slot * _K + j]],
                    rows_v.at[pl.ds((slot * _K + j) * _CHUNK, _CHUNK)],
                    gsem,
                )
                for j in range(_K)
            ]
            for c in copies:
                c.wait()
            pltpu.async_copy(
                rows_v.at[pl.ds(slot * _SLAB, _SLAB)],
                out_hbm.at[pl.ds(cbase * _CHUNK, _SLAB), pl.ds(0, EMBED_DIM)],
                wsem,
            )

        def drain_write(slot):
            wsem = wsem0 if slot == 0 else wsem1
            pltpu.make_async_copy(
                rows_v.at[pl.ds(slot * _SLAB, _SLAB)],
                out_hbm.at[pl.ds(0, _SLAB), pl.ds(0, EMBED_DIM)],
                wsem,
            ).wait()

        do_step(0, 0)
        do_step(1, 1)

        def body(t, carry):
            g = 2 * t + 2
            drain_write(0)
            do_step(g, 0)
            drain_write(1)
            do_step(g + 1, 1)
            return carry

        lax.fori_loop(0, (steps - 2) // 2, body, 0)

        drain_write(0)
        drain_write(1)

    return k


_TC_BLK = 4096


def _tc_slice(n):
    def body(i_ref, o_ref):
        o_ref[...] = i_ref[:, :EMBED_DIM]

    return pl.pallas_call(
        body,
        grid=(n // _TC_BLK,),
        in_specs=[pl.BlockSpec((_TC_BLK, _PADDED), lambda g: (g, 0))],
        out_specs=pl.BlockSpec((_TC_BLK, EMBED_DIM), lambda g: (g, 0)),
        out_shape=jax.ShapeDtypeStruct((n, EMBED_DIM), jnp.float32),
    )


def kernel(xs, table):
    b, s = xs.shape
    n = b * s
    idx2d = xs.reshape(n // _CHUNK, _CHUNK).astype(jnp.int32)
    out = _make_gather(n)(table, idx2d)
    return _tc_slice(n)(out).reshape(b, s, EMBED_DIM)


# speedup vs baseline: 2.0074x; 2.0074x over previous
import functools

import jax
import jax.numpy as jnp
from jax import lax
from jax.experimental import pallas as pl
from jax.experimental.pallas import tpu as pltpu
from jax.experimental.pallas import tpu_sc as plsc

VOCAB = 1000
EMBED_DIM = 64
_PADDED = 128

_NC = 2
_NS = 16
_NW = _NC * _NS

_CHUNK = 128
_K = 5
_SLAB = _K * _CHUNK


def _make_gather(n_idx: int):
    per_w = n_idx // _NW
    chunks_per_w = per_w // _CHUNK
    steps = chunks_per_w // _K
    assert per_w * _NW == n_idx
    assert steps * _K == chunks_per_w and chunks_per_w * _CHUNK == per_w
    assert steps >= 2 and steps % 2 == 0

    mesh = plsc.VectorSubcoreMesh(core_axis_name="c", subcore_axis_name="s")

    @functools.partial(
        pl.kernel,
        mesh=mesh,
        out_type=jax.ShapeDtypeStruct((n_idx, _PADDED), jnp.float32),
        scratch_types=[
            pltpu.VMEM_SHARED((VOCAB, EMBED_DIM), jnp.float32),
            pltpu.VMEM((2 * _K, _CHUNK), jnp.int32),
            pltpu.VMEM((2 * _SLAB, EMBED_DIM), jnp.float32),
            pltpu.SemaphoreType.DMA,
            pltpu.SemaphoreType.DMA,
            pltpu.SemaphoreType.DMA,
        ],
        compiler_params=pltpu.CompilerParams(use_tc_tiling_on_sc=False),
    )
    def k(table_hbm, idx_hbm, out_hbm, table_sh, idx_v, rows_v, gsem, wsem0,
          wsem1):
        cid = lax.axis_index("c")
        sid = lax.axis_index("s")
        wid = sid * _NC + cid

        @pl.when(sid == 0)
        def _stage():
            pltpu.sync_copy(table_hbm, table_sh)

        plsc.subcore_barrier()

        def do_step(g, slot):
            wsem = wsem0 if slot == 0 else wsem1
            cbase = wid * chunks_per_w + g * _K
            pltpu.sync_copy(
                idx_hbm.at[pl.ds(cbase, _K)],
                idx_v.at[pl.ds(slot * _K, _K)],
            )
            copies = [
                pltpu.async_copy(
                    table_sh.at[idx_v.at[slot * _K + j]],
                    rows_v.at[pl.ds((slot * _K + j) * _CHUNK, _CHUNK)],
                    gsem,
                )
                for j in range(_K)
            ]
            for c in copies:
                c.wait()
            pltpu.async_copy(
                rows_v.at[pl.ds(slot * _SLAB, _SLAB)],
                out_hbm.at[pl.ds(cbase * _CHUNK, _SLAB), pl.ds(0, EMBED_DIM)],
                wsem,
            )

        def drain_write(slot):
            wsem = wsem0 if slot == 0 else wsem1
            pltpu.make_async_copy(
                rows_v.at[pl.ds(slot * _SLAB, _SLAB)],
                out_hbm.at[pl.ds(0, _SLAB), pl.ds(0, EMBED_DIM)],
                wsem,
            ).wait()

        do_step(0, 0)
        do_step(1, 1)

        def body(t, carry):
            g = 2 * t + 2
            drain_write(0)
            do_step(g, 0)
            drain_write(1)
            do_step(g + 1, 1)
            return carry

        lax.fori_loop(0, (steps - 2) // 2, body, 0)

        drain_write(0)
        drain_write(1)

    return k


def kernel(xs, table):
    b, s = xs.shape
    n = b * s
    idx2d = xs.reshape(n // _CHUNK, _CHUNK).astype(jnp.int32)
    out = _make_gather(n)(table, idx2d)
    return out.reshape(b, s, _PADDED)[:, :, :EMBED_DIM]


# speedup vs baseline: 2.1503x; 1.0712x over previous
import functools

import jax
import jax.numpy as jnp
from jax import lax
from jax.experimental import pallas as pl
from jax.experimental.pallas import tpu as pltpu
from jax.experimental.pallas import tpu_sc as plsc

VOCAB = 1000
EMBED_DIM = 64
_PADDED = 128

_NC = 2
_NS = 16
_NW = _NC * _NS

_CHUNK = 128
_K = 5
_SLAB = _K * _CHUNK


def _make_gather(n_idx: int):
    per_w = n_idx // _NW
    chunks_per_w = per_w // _CHUNK
    steps = chunks_per_w // _K
    assert per_w * _NW == n_idx
    assert steps * _K == chunks_per_w and chunks_per_w * _CHUNK == per_w
    assert steps >= 2 and steps % 2 == 0

    mesh = plsc.VectorSubcoreMesh(core_axis_name="c", subcore_axis_name="s")

    @functools.partial(
        pl.kernel,
        mesh=mesh,
        out_type=jax.ShapeDtypeStruct((n_idx, _PADDED), jnp.float32),
        scratch_types=[
            pltpu.VMEM_SHARED((VOCAB, EMBED_DIM), jnp.float32),
            pltpu.VMEM((2 * _K, _CHUNK), jnp.int32),
            pltpu.VMEM((2 * _SLAB, EMBED_DIM), jnp.float32),
            pltpu.SemaphoreType.DMA,
            pltpu.SemaphoreType.DMA,
            pltpu.SemaphoreType.DMA,
            pltpu.SemaphoreType.DMA,
        ],
        compiler_params=pltpu.CompilerParams(use_tc_tiling_on_sc=False),
    )
    def k(table_hbm, idx_hbm, out_hbm, table_sh, idx_v, rows_v, gsem, wsem0,
          wsem1, psem):
        cid = lax.axis_index("c")
        sid = lax.axis_index("s")
        wid = sid * _NC + cid

        @pl.when(sid == 0)
        def _stage():
            pltpu.sync_copy(table_hbm, table_sh)

        plsc.subcore_barrier()

        def load_idx(g, slot):
            cb = jnp.minimum(wid * chunks_per_w + g * _K,
                             (wid + 1) * chunks_per_w - _K)
            pltpu.async_copy(
                idx_hbm.at[pl.ds(cb, _K)],
                idx_v.at[pl.ds(slot * _K, _K)],
                psem,
            )

        def wait_idx(slot):
            pltpu.make_async_copy(
                idx_hbm.at[pl.ds(0, _K)],
                idx_v.at[pl.ds(slot * _K, _K)],
                psem,
            ).wait()

        def do_step(g, slot):
            wsem = wsem0 if slot == 0 else wsem1
            cbase = wid * chunks_per_w + g * _K
            wait_idx(slot)
            load_idx(g + 1, 1 - slot)
            copies = [
                pltpu.async_copy(
                    table_sh.at[idx_v.at[slot * _K + j]],
                    rows_v.at[pl.ds((slot * _K + j) * _CHUNK, _CHUNK)],
                    gsem,
                )
                for j in range(_K)
            ]
            for c in copies:
                c.wait()
            pltpu.async_copy(
                rows_v.at[pl.ds(slot * _SLAB, _SLAB)],
                out_hbm.at[pl.ds(cbase * _CHUNK, _SLAB), pl.ds(0, EMBED_DIM)],
                wsem,
            )

        def drain_write(slot):
            wsem = wsem0 if slot == 0 else wsem1
            pltpu.make_async_copy(
                rows_v.at[pl.ds(slot * _SLAB, _SLAB)],
                out_hbm.at[pl.ds(0, _SLAB), pl.ds(0, EMBED_DIM)],
                wsem,
            ).wait()

        load_idx(0, 0)
        do_step(0, 0)
        do_step(1, 1)

        def body(t, carry):
            g = 2 * t + 2
            drain_write(0)
            do_step(g, 0)
            drain_write(1)
            do_step(g + 1, 1)
            return carry

        lax.fori_loop(0, (steps - 2) // 2, body, 0)

        wait_idx(0)
        drain_write(0)
        drain_write(1)

    return k


def kernel(xs, table):
    b, s = xs.shape
    n = b * s
    idx2d = xs.reshape(n // _CHUNK, _CHUNK).astype(jnp.int32)
    out = _make_gather(n)(table, idx2d)
    return out.reshape(b, s, _PADDED)[:, :, :EMBED_DIM]
